# baseline (speedup 1.0000x reference)
import functools

import jax
import jax.numpy as jnp
from jax import lax
from jax.experimental import pallas as pl
from jax.experimental.pallas import tpu as pltpu
from jax.experimental.pallas import tpu_sc as plsc

N, C, H, W = 4, 96, 224, 224
HW = H * W
NPIX = N * HW
NWORK = 32
PPW = NPIX // NWORK
CHUNK = 32
NCHUNK = PPW // CHUNK
NPAIR = NCHUNK // 2
CW = C // 2
_HIMASK = -65536


def _floorf(v):
    vi = v.astype(jnp.int32)
    vf = vi.astype(jnp.float32)
    return jnp.where(vf > v, vf - 1.0, vf)


def _corners(ix, iy, masked):
    x0f = _floorf(ix)
    y0f = _floorf(iy)
    x1f = x0f + 1.0
    y1f = y0f + 1.0
    wx1 = ix - x0f
    wx0 = 1.0 - wx1
    wy1 = iy - y0f
    wy0 = 1.0 - wy1
    if masked:
        wx0 = wx0 * jnp.where((x0f >= 0.0) & (x0f <= W - 1.0), 1.0, 0.0)
        wx1 = wx1 * jnp.where((x1f >= 0.0) & (x1f <= W - 1.0), 1.0, 0.0)
        wy0 = wy0 * jnp.where((y0f >= 0.0) & (y0f <= H - 1.0), 1.0, 0.0)
        wy1 = wy1 * jnp.where((y1f >= 0.0) & (y1f <= H - 1.0), 1.0, 0.0)
    xi0 = jnp.clip(x0f.astype(jnp.int32), 0, W - 1)
    xi1 = jnp.clip(x1f.astype(jnp.int32), 0, W - 1)
    yi0 = jnp.clip(y0f.astype(jnp.int32), 0, H - 1)
    yi1 = jnp.clip(y1f.astype(jnp.int32), 0, H - 1)
    ws = (wy0 * wx0, wy0 * wx1, wy1 * wx0, wy1 * wx1)
    cols = (xi0, xi1, xi0, xi1)
    rows = (yi0, yi0, yi1, yi1)
    return ws, rows, cols


def _reflect(c):
    span = float(W - 1)
    a = jnp.abs(c)
    extra = lax.rem(a, span)
    flips = _floorf(a / span)
    even = lax.rem(flips, 2.0) == 0.0
    return jnp.where(even, extra, span - extra)


def _mode_coords(gx, gy, mode):
    if mode == 0:
        ix = (gx + 1.0) * 0.5 * (W - 1.0)
        iy = (gy + 1.0) * 0.5 * (H - 1.0)
    elif mode == 1:
        ix = ((gx + 1.0) * W - 1.0) * 0.5
        iy = ((gy + 1.0) * H - 1.0) * 0.5
        ix = jnp.clip(ix, 0.0, W - 1.0)
        iy = jnp.clip(iy, 0.0, H - 1.0)
    else:
        ix = jnp.clip(_reflect((gx + 1.0) * 0.5 * (W - 1.0)), 0.0, W - 1.0)
        iy = jnp.clip(_reflect((gy + 1.0) * 0.5 * (H - 1.0)), 0.0, H - 1.0)
    return ix, iy


@functools.cache
def _build_sc_kernel():
  mesh = plsc.VectorSubcoreMesh(core_axis_name="c", subcore_axis_name="s")

  @functools.partial(
    pl.kernel,
    mesh=mesh,
    compiler_params=pltpu.CompilerParams(use_tc_tiling_on_sc=False),
    out_type=[jax.ShapeDtypeStruct((NPIX, CW), jnp.int32)] * 3,
    scratch_types=(
        [pltpu.VMEM((2 * CHUNK,), jnp.float32)] * 2
        + [pltpu.VMEM((4 * CHUNK,), jnp.int32)] * 6
        + [pltpu.VMEM((5 * CHUNK,), jnp.float32)] * 6
        + [pltpu.VMEM((4 * CHUNK, CW), jnp.int32)] * 6
        + [pltpu.VMEM((CHUNK, CW), jnp.int32)] * 6
        + [pltpu.SemaphoreType.DMA] * 13
    ),
  )
  def _grid_sample_sc(xtp, gxa, gya, o0, o1, o2,
                      gx_v, gy_v,
                      i00, i01, i02, i10, i11, i12,
                      w00, w01, w02, w10, w11, w12,
                      r00, r01, r02, r10, r11, r12,
                      v00, v01, v02, v10, v11, v12,
                      gsem, gs00, gs01, gs02, gs10, gs11, gs12,
                      os00, os01, os02, os10, os11, os12):
    wid = lax.axis_index("s") * 2 + lax.axis_index("c")
    img = wid // 8
    rowbase = img * HW
    pixbase = wid * PPW
    inbase = (wid % 8) * PPW

    idx_b = ((i00, i01, i02), (i10, i11, i12))
    w_b = ((w00, w01, w02), (w10, w11, w12))
    rows_b = ((r00, r01, r02), (r10, r11, r12))
    out_b = ((v00, v01, v02), (v10, v11, v12))
    gsems = ((gs00, gs01, gs02), (gs10, gs11, gs12))
    osems = ((os00, os01, os02), (os10, os11, os12))
    out_hbms = (o0, o1, o2)

    def load_grid(pair):
        gbase = pixbase + pair * (2 * CHUNK)
        pltpu.async_copy(gxa.at[pl.ds(gbase, 2 * CHUNK)], gx_v, gsem)
        pltpu.async_copy(gya.at[pl.ds(gbase, 2 * CHUNK)], gy_v, gsem)

    def wait_grid():
        pltpu.make_async_copy(gxa.at[pl.ds(0, 2 * CHUNK)], gx_v, gsem).wait()
        pltpu.make_async_copy(gya.at[pl.ds(0, 2 * CHUNK)], gy_v, gsem).wait()

    def prep_pair(pair):
        for s in range(2):
            for g in range(2):
                gx = gx_v[pl.ds(s * CHUNK + g * 16, 16)]
                gy = gy_v[pl.ds(s * CHUNK + g * 16, 16)]
                for m in range(3):
                    ix, iy = _mode_coords(gx, gy, m)
                    ws, rr, cc = _corners(ix, iy, masked=(m == 0))
                    for c in range(4):
                        pos = pl.ds(c * CHUNK + g * 16, 16)
                        idx_b[s][m][pos] = rowbase + rr[c] * W + cc[c]
                        w_b[s][m][pos] = ws[c]
        for s in range(2):
            for m in range(3):
                pltpu.async_copy(xtp.at[idx_b[s][m]], rows_b[s][m], gsems[s][m])

    def combine(rows, w_r, out_v):
        def px(p, carry):
            wsc = [w_r[pl.ds(c * CHUNK + p, 16)][0] for c in range(4)]
            for cg in range(C // 32):
                acc_lo = None
                acc_hi = None
                for c in range(4):
                    wrd = rows[c * CHUNK + p, pl.ds(cg * 16, 16)]
                    lo = lax.bitcast_convert_type(wrd << 16, jnp.float32)
                    hi = lax.bitcast_convert_type(wrd & _HIMASK, jnp.float32)
                    if acc_lo is None:
                        acc_lo = lo * wsc[c]
                        acc_hi = hi * wsc[c]
                    else:
                        acc_lo = acc_lo + lo * wsc[c]
                        acc_hi = acc_hi + hi * wsc[c]
                lo_i = lax.bitcast_convert_type(acc_lo, jnp.int32)
                hi_i = lax.bitcast_convert_type(acc_hi, jnp.int32)
                packed = lax.shift_right_logical(lo_i, 16) | (hi_i & _HIMASK)
                out_v[p, pl.ds(cg * 16, 16)] = packed
            return carry

        lax.fori_loop(0, CHUNK, px, 0, unroll=8)

    load_grid(0)
    wait_grid()
    prep_pair(0)
    load_grid(1)

    def body(k, carry):
        for s in range(2):
            base = pixbase + (2 * k + s) * CHUNK
            for m in range(3):
                @pl.when(k > 0)
                def _():
                    pltpu.make_async_copy(
                        out_b[s][m], out_hbms[m].at[pl.ds(0, CHUNK)],
                        osems[s][m]).wait()
                pltpu.make_async_copy(
                    xtp.at[idx_b[s][m]], rows_b[s][m], gsems[s][m]).wait()
                combine(rows_b[s][m], w_b[s][m], out_b[s][m])
                pltpu.async_copy(
                    out_b[s][m], out_hbms[m].at[pl.ds(base, CHUNK)],
                    osems[s][m])
        wait_grid()
        prep_pair(jnp.minimum(k + 1, NPAIR - 1))
        load_grid(jnp.minimum(k + 2, NPAIR - 1))
        return carry

    lax.fori_loop(0, NPAIR, body, 0)

    wait_grid()
    for s in range(2):
        for m in range(3):
            pltpu.make_async_copy(
                xtp.at[idx_b[s][m]], rows_b[s][m], gsems[s][m]).wait()
            pltpu.make_async_copy(
                out_b[s][m], out_hbms[m].at[pl.ds(0, CHUNK)],
                osems[s][m]).wait()

  return _grid_sample_sc


def kernel(x, T):
    xb = x.astype(jnp.bfloat16)
    xt = xb.transpose(0, 2, 3, 1).reshape(NPIX, 3, 2, 16)
    xt = xt.transpose(0, 1, 3, 2).reshape(NPIX, CW, 2)
    xtp = lax.bitcast_convert_type(xt, jnp.int32)
    g = T.reshape(NPIX, 2)
    o0, o1, o2 = _build_sc_kernel()(xtp, g[:, 0], g[:, 1])

    def back(o):
        ob = lax.bitcast_convert_type(o, jnp.bfloat16)
        ob = ob.reshape(NPIX, 3, 16, 2).transpose(0, 1, 3, 2)
        ob = ob.reshape(N, H, W, C).transpose(0, 3, 1, 2)
        return ob.astype(jnp.float32)

    return (back(o0), back(o1), back(o2))


# speedup vs baseline: 1.4998x; 1.4998x over previous
import functools

import jax
import jax.numpy as jnp
from jax import lax
from jax.experimental import pallas as pl
from jax.experimental.pallas import tpu as pltpu
from jax.experimental.pallas import tpu_sc as plsc

N, C, H, W = 4, 96, 224, 224
HW = H * W
NPIX = N * HW
NWORK = 32
PPW = NPIX // NWORK
CHUNK = 32
NCHUNK = PPW // CHUNK
NPAIR = NCHUNK // 2
CW = C // 2
_HIMASK = -65536


def _floorf(v):
    vi = v.astype(jnp.int32)
    vf = vi.astype(jnp.float32)
    return jnp.where(vf > v, vf - 1.0, vf)


def _corners(ix, iy, masked):
    x0f = _floorf(ix)
    y0f = _floorf(iy)
    x1f = x0f + 1.0
    y1f = y0f + 1.0
    wx1 = ix - x0f
    wx0 = 1.0 - wx1
    wy1 = iy - y0f
    wy0 = 1.0 - wy1
    if masked:
        wx0 = wx0 * jnp.where((x0f >= 0.0) & (x0f <= W - 1.0), 1.0, 0.0)
        wx1 = wx1 * jnp.where((x1f >= 0.0) & (x1f <= W - 1.0), 1.0, 0.0)
        wy0 = wy0 * jnp.where((y0f >= 0.0) & (y0f <= H - 1.0), 1.0, 0.0)
        wy1 = wy1 * jnp.where((y1f >= 0.0) & (y1f <= H - 1.0), 1.0, 0.0)
    xi0 = jnp.clip(x0f.astype(jnp.int32), 0, W - 1)
    xi1 = jnp.clip(x1f.astype(jnp.int32), 0, W - 1)
    yi0 = jnp.clip(y0f.astype(jnp.int32), 0, H - 1)
    yi1 = jnp.clip(y1f.astype(jnp.int32), 0, H - 1)
    ws = (wy0 * wx0, wy0 * wx1, wy1 * wx0, wy1 * wx1)
    cols = (xi0, xi1, xi0, xi1)
    rows = (yi0, yi0, yi1, yi1)
    return ws, rows, cols


def _reflect(c):
    span = float(W - 1)
    a = jnp.abs(c)
    extra = lax.rem(a, span)
    flips = _floorf(a / span)
    even = lax.rem(flips, 2.0) == 0.0
    return jnp.where(even, extra, span - extra)


def _mode_coords(gx, gy, mode):
    if mode == 0:
        ix = (gx + 1.0) * 0.5 * (W - 1.0)
        iy = (gy + 1.0) * 0.5 * (H - 1.0)
    elif mode == 1:
        ix = ((gx + 1.0) * W - 1.0) * 0.5
        iy = ((gy + 1.0) * H - 1.0) * 0.5
        ix = jnp.clip(ix, 0.0, W - 1.0)
        iy = jnp.clip(iy, 0.0, H - 1.0)
    else:
        ix = jnp.clip(_reflect((gx + 1.0) * 0.5 * (W - 1.0)), 0.0, W - 1.0)
        iy = jnp.clip(_reflect((gy + 1.0) * 0.5 * (H - 1.0)), 0.0, H - 1.0)
    return ix, iy


@functools.cache
def _build_sc_kernel():
  mesh = plsc.VectorSubcoreMesh(core_axis_name="c", subcore_axis_name="s")

  @functools.partial(
    pl.kernel,
    mesh=mesh,
    compiler_params=pltpu.CompilerParams(use_tc_tiling_on_sc=False),
    out_type=[jax.ShapeDtypeStruct((NPIX, C), jnp.float32)] * 3,
    scratch_types=(
        [pltpu.VMEM((2 * CHUNK,), jnp.float32)] * 2
        + [pltpu.VMEM((4 * CHUNK,), jnp.int32)] * 6
        + [pltpu.VMEM((5 * CHUNK,), jnp.float32)] * 6
        + [pltpu.VMEM((4 * CHUNK, CW), jnp.int32)] * 6
        + [pltpu.VMEM((CHUNK, C), jnp.float32)] * 6
        + [pltpu.SemaphoreType.DMA] * 13
    ),
  )
  def _grid_sample_sc(xtp, gxa, gya, o0, o1, o2,
                      gx_v, gy_v,
                      i00, i01, i02, i10, i11, i12,
                      w00, w01, w02, w10, w11, w12,
                      r00, r01, r02, r10, r11, r12,
                      v00, v01, v02, v10, v11, v12,
                      gsem, gs00, gs01, gs02, gs10, gs11, gs12,
                      os00, os01, os02, os10, os11, os12):
    wid = lax.axis_index("s") * 2 + lax.axis_index("c")
    img = wid // 8
    rowbase = img * HW
    pixbase = wid * PPW
    inbase = (wid % 8) * PPW

    idx_b = ((i00, i01, i02), (i10, i11, i12))
    w_b = ((w00, w01, w02), (w10, w11, w12))
    rows_b = ((r00, r01, r02), (r10, r11, r12))
    out_b = ((v00, v01, v02), (v10, v11, v12))
    gsems = ((gs00, gs01, gs02), (gs10, gs11, gs12))
    osems = ((os00, os01, os02), (os10, os11, os12))
    out_hbms = (o0, o1, o2)

    def load_grid(pair):
        gbase = pixbase + pair * (2 * CHUNK)
        pltpu.async_copy(gxa.at[pl.ds(gbase, 2 * CHUNK)], gx_v, gsem)
        pltpu.async_copy(gya.at[pl.ds(gbase, 2 * CHUNK)], gy_v, gsem)

    def wait_grid():
        pltpu.make_async_copy(gxa.at[pl.ds(0, 2 * CHUNK)], gx_v, gsem).wait()
        pltpu.make_async_copy(gya.at[pl.ds(0, 2 * CHUNK)], gy_v, gsem).wait()

    def prep_pair(pair):
        for s in range(2):
            for g in range(2):
                gx = gx_v[pl.ds(s * CHUNK + g * 16, 16)]
                gy = gy_v[pl.ds(s * CHUNK + g * 16, 16)]
                for m in range(3):
                    ix, iy = _mode_coords(gx, gy, m)
                    ws, rr, cc = _corners(ix, iy, masked=(m == 0))
                    for c in range(4):
                        pos = pl.ds(c * CHUNK + g * 16, 16)
                        idx_b[s][m][pos] = rowbase + rr[c] * W + cc[c]
                        w_b[s][m][pos] = ws[c]
        for s in range(2):
            for m in range(3):
                pltpu.async_copy(xtp.at[idx_b[s][m]], rows_b[s][m], gsems[s][m])

    def combine(rows, w_r, out_v):
        def px(p, carry):
            wsc = [w_r[pl.ds(c * CHUNK + p, 16)][0] for c in range(4)]
            for cg in range(C // 32):
                acc_lo = None
                acc_hi = None
                for c in range(4):
                    wrd = rows[c * CHUNK + p, pl.ds(cg * 16, 16)]
                    lo = lax.bitcast_convert_type(wrd << 16, jnp.float32)
                    hi = lax.bitcast_convert_type(wrd & _HIMASK, jnp.float32)
                    if acc_lo is None:
                        acc_lo = lo * wsc[c]
                        acc_hi = hi * wsc[c]
                    else:
                        acc_lo = acc_lo + lo * wsc[c]
                        acc_hi = acc_hi + hi * wsc[c]
                out_v[p, pl.ds(cg * 32, 16)] = acc_lo
                out_v[p, pl.ds(cg * 32 + 16, 16)] = acc_hi
            return carry

        lax.fori_loop(0, CHUNK, px, 0, unroll=8)

    load_grid(0)
    wait_grid()
    prep_pair(0)
    load_grid(1)

    def body(k, carry):
        for s in range(2):
            base = pixbase + (2 * k + s) * CHUNK
            for m in range(3):
                @pl.when(k > 0)
                def _():
                    pltpu.make_async_copy(
                        out_b[s][m], out_hbms[m].at[pl.ds(0, CHUNK)],
                        osems[s][m]).wait()
                pltpu.make_async_copy(
                    xtp.at[idx_b[s][m]], rows_b[s][m], gsems[s][m]).wait()
                combine(rows_b[s][m], w_b[s][m], out_b[s][m])
                pltpu.async_copy(
                    out_b[s][m], out_hbms[m].at[pl.ds(base, CHUNK)],
                    osems[s][m])
        wait_grid()
        prep_pair(jnp.minimum(k + 1, NPAIR - 1))
        load_grid(jnp.minimum(k + 2, NPAIR - 1))
        return carry

    lax.fori_loop(0, NPAIR, body, 0)

    wait_grid()
    for s in range(2):
        for m in range(3):
            pltpu.make_async_copy(
                xtp.at[idx_b[s][m]], rows_b[s][m], gsems[s][m]).wait()
            pltpu.make_async_copy(
                out_b[s][m], out_hbms[m].at[pl.ds(0, CHUNK)],
                osems[s][m]).wait()

  return _grid_sample_sc


def kernel(x, T):
    xb = x.astype(jnp.bfloat16)
    xt = xb.transpose(0, 2, 3, 1).reshape(NPIX, 3, 2, 16)
    xt = xt.transpose(0, 1, 3, 2).reshape(NPIX, CW, 2)
    xtp = lax.bitcast_convert_type(xt, jnp.int32)
    g = T.reshape(NPIX, 2)
    o0, o1, o2 = _build_sc_kernel()(xtp, g[:, 0], g[:, 1])

    def back(o):
        return o.reshape(N, H, W, C).transpose(0, 3, 1, 2)

    return (back(o0), back(o1), back(o2))
